# baseline (speedup 1.0000x reference)
import functools

import jax
import jax.numpy as jnp
from jax import lax
from jax.experimental import pallas as pl
from jax.experimental.pallas import tpu as pltpu
from jax.experimental.pallas import tpu_sc as plsc

_NW = 32
_CW = 128
_L = 16
_TBLK = 2048
_H = 50176
_PBLK = 512


def _fmt_pairs(x_t, H):
    D, N = x_t.shape
    nblk = H // _PBLK

    def body(lo_ref, hi_ref, out_ref):
        eye = jnp.eye(D, dtype=jnp.float32)
        lo = jax.lax.dot_general(
            lo_ref[...], eye, (((0,), (0,)), ((), ())),
            precision=jax.lax.Precision.DEFAULT)
        hi = jax.lax.dot_general(
            hi_ref[...], eye, (((0,), (0,)), ((), ())),
            precision=jax.lax.Precision.DEFAULT)
        out_ref[...] = jnp.concatenate([lo, hi], axis=1)

    nb_half = H // _PBLK

    return pl.pallas_call(
        body,
        grid=(nblk,),
        in_specs=[
            pl.BlockSpec((D, _PBLK), lambda i: (0, i)),
            pl.BlockSpec((D, _PBLK), lambda i: (0, i + nb_half)),
        ],
        out_specs=pl.BlockSpec((_PBLK, 2 * D), lambda i: (i, 0)),
        out_shape=jax.ShapeDtypeStruct((H, 2 * D), jnp.float32),
    )(x_t, x_t)


def _fmt_dup(x_t):
    D, N = x_t.shape
    nblk = (N + _TBLK - 1) // _TBLK

    def body(in_ref, out_ref):
        eye = jnp.eye(D, dtype=jnp.float32)
        eye2 = jnp.concatenate([eye, eye], axis=1)
        out_ref[...] = jax.lax.dot_general(
            in_ref[...], eye2, (((0,), (0,)), ((), ())),
            precision=jax.lax.Precision.DEFAULT)

    return pl.pallas_call(
        body,
        grid=(nblk,),
        in_specs=[pl.BlockSpec((D, _TBLK), lambda i: (0, i))],
        out_specs=pl.BlockSpec((_TBLK, 2 * D), lambda i: (i, 0)),
        out_shape=jax.ShapeDtypeStruct((N, 2 * D), jnp.float32),
    )(x_t)


def kernel(embeddings, labels, centers):
    B, D = embeddings.shape
    ch = B // (_NW * _CW)
    bw = _CW * ch
    DP = 2 * D
    nring = 2

    ctr_r = _fmt_pairs(centers.T, _H)
    emb_r = _fmt_dup(embeddings.T)
    labels = labels.astype(jnp.int32)
    hi = labels >= _H
    idx = jnp.where(hi, labels - _H, labels)
    sel = jnp.broadcast_to(
        hi.astype(jnp.float32)[:, None], (B, _L)).reshape(B, _L)

    mesh = plsc.VectorSubcoreMesh(core_axis_name="c", subcore_axis_name="s")

    @functools.partial(
        pl.kernel,
        mesh=mesh,
        compiler_params=pltpu.CompilerParams(
            use_tc_tiling_on_sc=True, needs_layout_passes=False
        ),
        out_type=jax.ShapeDtypeStruct((_NW * _L,), jnp.float32),
        scratch_types=[
            pltpu.VMEM((bw,), jnp.int32),
            pltpu.VMEM((bw // 2, DP), jnp.float32),
            pltpu.VMEM((bw // 2, _L), jnp.float32),
            pltpu.VMEM((nring * _CW, DP), jnp.float32),
            pltpu.VMEM((_L,), jnp.float32),
            pltpu.SemaphoreType.DMA,
            pltpu.SemaphoreType.DMA,
            pltpu.SemaphoreType.DMA,
            pltpu.SemaphoreType.DMA,
        ],
    )
    def sc_kernel(emb_hbm, idx_hbm, sel_hbm, ctr_hbm, out_hbm,
                  idx_v, emb_v, sel_v, ctr_v, acc_v,
                  sem_e, sem_s, sem_g0, sem_g1):
        wid = lax.axis_index("s") * 2 + lax.axis_index("c")
        base = pl.multiple_of(wid * bw, bw)

        emb_dma = pltpu.async_copy(
            emb_hbm.at[pl.ds(base, bw // 2)], emb_v, sem_e)
        sel_dma = pltpu.async_copy(
            sel_hbm.at[pl.ds(base, bw // 2)], sel_v, sem_s)
        pltpu.sync_copy(idx_hbm.at[pl.ds(base, bw)], idx_v)
        sems = [sem_g0, sem_g1]

        def fire(j):
            return pltpu.async_copy(
                ctr_hbm.at[idx_v.at[pl.ds(j * _CW, _CW)]],
                ctr_v.at[pl.ds((j % nring) * _CW, _CW)],
                sems[j % nring],
            )

        gathers = [fire(0), fire(1)]
        emb_dma.wait()
        sel_dma.wait()

        ngrp = _CW // 8

        ngrp_half = (bw // 2) // 8

        def make_grp_body(slot):
            def grp_body(g, acc):
                ebase = pl.multiple_of((g % ngrp_half) * 8, 8)
                cbase = pl.multiple_of((slot * ngrp + g % ngrp) * 8, 8)
                ev = emb_v.at[pl.ds(ebase, 8)]
                sv = sel_v.at[pl.ds(ebase, 8)]
                cv = ctr_v.at[pl.ds(cbase, 8)]
                for k in range(8):
                    p = sv[k, :]
                    for c in range(D // _L):
                        e = ev[k, pl.ds(c * _L, _L)]
                        clo = cv[k, pl.ds(c * _L, _L)]
                        chi = cv[k, pl.ds(D + c * _L, _L)]
                        cs = clo + p * (chi - clo)
                        d = e - cs
                        acc = acc + d * d
                return acc
            return grp_body

        acc = jnp.zeros((_L,), jnp.float32)
        for j in range(ch):
            gathers[j].wait()
            acc = lax.fori_loop(j * ngrp, (j + 1) * ngrp,
                                make_grp_body(j % nring), acc)
            if j + nring < ch:
                gathers.append(fire(j + nring))
            if j == ch // 2 - 1:
                e2 = pltpu.async_copy(
                    emb_hbm.at[pl.ds(base + bw // 2, bw // 2)], emb_v, sem_e)
                s2 = pltpu.async_copy(
                    sel_hbm.at[pl.ds(base + bw // 2, bw // 2)], sel_v, sem_s)
                e2.wait()
                s2.wait()

        acc_v[...] = acc
        pltpu.sync_copy(acc_v, out_hbm.at[pl.ds(wid * _L, _L)])

    partials = sc_kernel(emb_r, idx, sel, ctr_r)
    return partials.sum() / B


# speedup vs baseline: 1.7578x; 1.7578x over previous
import functools

import jax
import jax.numpy as jnp
from jax import lax
from jax.experimental import pallas as pl
from jax.experimental.pallas import tpu as pltpu
from jax.experimental.pallas import tpu_sc as plsc

_NW = 32
_CW = 128
_L = 16
_TBLK = 4096


def _fmt_rows(x_t):
    D, N = x_t.shape
    nblk = (N + _TBLK - 1) // _TBLK

    def body(in_ref, out_ref):
        eye = jnp.eye(D, dtype=jnp.float32)
        eye2 = jnp.concatenate([eye, eye], axis=1)
        out_ref[...] = jax.lax.dot_general(
            in_ref[...], eye2, (((0,), (0,)), ((), ())),
            precision=jax.lax.Precision.DEFAULT,
        )

    return pl.pallas_call(
        body,
        grid=(nblk,),
        in_specs=[pl.BlockSpec((D, _TBLK), lambda i: (0, i))],
        out_specs=pl.BlockSpec((_TBLK, 2 * D), lambda i: (i, 0)),
        out_shape=jax.ShapeDtypeStruct((N, 2 * D), jnp.float32),
    )(x_t)


def kernel(embeddings, labels, centers):
    B, D = embeddings.shape
    ch = B // (_NW * _CW)
    bw = _CW * ch
    DP = 2 * D
    nring = 2

    ctr_r = _fmt_rows(centers.T)
    idx = labels.astype(jnp.int32)

    mesh = plsc.VectorSubcoreMesh(core_axis_name="c", subcore_axis_name="s")

    @functools.partial(
        pl.kernel,
        mesh=mesh,
        compiler_params=pltpu.CompilerParams(
            use_tc_tiling_on_sc=True, needs_layout_passes=False
        ),
        out_type=jax.ShapeDtypeStruct((_NW * _L,), jnp.float32),
        scratch_types=[
            pltpu.VMEM((bw,), jnp.int32),
            pltpu.VMEM((bw, D), jnp.float32),
            pltpu.VMEM((nring * _CW, DP), jnp.float32),
            pltpu.VMEM((_L,), jnp.float32),
            pltpu.SemaphoreType.DMA,
            pltpu.SemaphoreType.DMA,
            pltpu.SemaphoreType.DMA,
        ],
    )
    def sc_kernel(emb_hbm, idx_hbm, ctr_hbm, out_hbm,
                  idx_v, emb_v, ctr_v, acc_v, sem_e, sem_g0, sem_g1):
        wid = lax.axis_index("s") * 2 + lax.axis_index("c")
        base = pl.multiple_of(wid * bw, bw)

        emb_dma = pltpu.async_copy(emb_hbm.at[pl.ds(base, bw)], emb_v, sem_e)
        pltpu.sync_copy(idx_hbm.at[pl.ds(base, bw)], idx_v)
        sems = [sem_g0, sem_g1]

        def fire(j):
            return pltpu.async_copy(
                ctr_hbm.at[idx_v.at[pl.ds(j * _CW, _CW)]],
                ctr_v.at[pl.ds((j % nring) * _CW, _CW)],
                sems[j % nring],
            )

        gathers = [fire(0), fire(1)]
        emb_dma.wait()

        ngrp = _CW // 8

        def make_grp_body(slot):
            def grp_body(g, acc):
                ebase = pl.multiple_of(g * 8, 8)
                cbase = pl.multiple_of((slot * ngrp + g % ngrp) * 8, 8)
                ev = emb_v.at[pl.ds(ebase, 8)]
                cv = ctr_v.at[pl.ds(cbase, 8)]
                for k in range(8):
                    for c in range(D // _L):
                        e = ev[k, pl.ds(c * _L, _L)]
                        t = cv[k, pl.ds(c * _L, _L)]
                        d = e - t
                        acc = acc + d * d
                return acc
            return grp_body

        acc = jnp.zeros((_L,), jnp.float32)
        for j in range(ch):
            gathers[j].wait()
            acc = lax.fori_loop(j * ngrp, (j + 1) * ngrp,
                                make_grp_body(j % nring), acc)
            if j + nring < ch:
                gathers.append(fire(j + nring))

        acc_v[...] = acc
        pltpu.sync_copy(acc_v, out_hbm.at[pl.ds(wid * _L, _L)])

    partials = sc_kernel(embeddings, idx, ctr_r)
    return partials.sum() / B
